# baseline (speedup 1.0000x reference)
import jax
import jax.numpy as jnp
from jax.experimental import pallas as pl
from jax.experimental.pallas import tpu as pltpu

_K = 1024
_D = 256
_BETA = 0.25
_NB = 16
_T = 1024
_NTOK = _NB * _T


def _vq_body(x_ref, cb_ref, loss_ref, q_ref, perp_ref, enc_ref,
             cnt_ref, acc_ref):
    b = pl.program_id(0)

    @pl.when(b == 0)
    def _init():
        cnt_ref[...] = jnp.zeros_like(cnt_ref)
        acc_ref[0] = 0.0

    x = x_ref[0]
    fb = jnp.transpose(x)
    cb = cb_ref[...]

    xsq = jnp.sum(fb * fb, axis=1, keepdims=True)
    cnorm = jnp.sum(cb * cb, axis=1)
    mm = jax.lax.dot_general(fb, cb, (((1,), (1,)), ((), ())),
                             preferred_element_type=jnp.float32)
    d = (xsq + cnorm[None, :]) - 2.0 * mm

    m = jnp.min(d, axis=1, keepdims=True)
    kio = jax.lax.broadcasted_iota(jnp.int32, (_T, _K), 1)
    idx = jnp.min(jnp.where(d == m, kio, _K), axis=1, keepdims=True)

    e = (kio == idx).astype(jnp.float32)
    enc_ref[...] = e
    cnt_ref[...] += jnp.sum(e, axis=0, keepdims=True)

    q = jax.lax.dot_general(cb, e, (((0,), (1,)), ((), ())),
                            preferred_element_type=jnp.float32)
    q_ref[0] = x + (q - x)

    acc_ref[0] += jnp.sum(m)

    @pl.when(b == _NB - 1)
    def _fin():
        mse = acc_ref[0] / (_NTOK * _D)
        loss_ref[...] = jnp.reshape((1.0 + _BETA) * mse, (1, 1))
        avg = cnt_ref[...] * (1.0 / _NTOK)
        ent = jnp.sum(avg * jnp.log(avg + 1e-10))
        perp_ref[...] = jnp.reshape(jnp.exp(-ent), (1, 1))


def _vq_call(x, codebook):
    return pl.pallas_call(
        _vq_body,
        grid=(_NB,),
        in_specs=[
            pl.BlockSpec((1, _D, _T), lambda b: (b, 0, 0)),
            pl.BlockSpec((_K, _D), lambda b: (0, 0)),
        ],
        out_specs=[
            pl.BlockSpec((1, 1), lambda b: (0, 0)),
            pl.BlockSpec((1, _D, _T), lambda b: (b, 0, 0)),
            pl.BlockSpec((1, 1), lambda b: (0, 0)),
            pl.BlockSpec((_T, _K), lambda b: (b, 0)),
        ],
        out_shape=[
            jax.ShapeDtypeStruct((1, 1), jnp.float32),
            jax.ShapeDtypeStruct((_NB, _D, _T), jnp.float32),
            jax.ShapeDtypeStruct((1, 1), jnp.float32),
            jax.ShapeDtypeStruct((_NTOK, _K), jnp.float32),
        ],
        scratch_shapes=[
            pltpu.VMEM((1, _K), jnp.float32),
            pltpu.SMEM((1,), jnp.float32),
        ],
        compiler_params=pltpu.CompilerParams(
            dimension_semantics=("arbitrary",),
        ),
    )(x, codebook)


def kernel(encoder_output, codebook):
    b, c, h, w = encoder_output.shape
    x = encoder_output.reshape(b, c, h * w)
    loss, q_st, perp, enc = _vq_call(x, codebook)
    return (loss[0, 0], q_st.reshape(b, c, h, w), perp[0, 0], enc)


# speedup vs baseline: 1.5494x; 1.5494x over previous
import jax
import jax.numpy as jnp
from jax.experimental import pallas as pl
from jax.experimental.pallas import tpu as pltpu

_K = 1024
_D = 256
_BETA = 0.25
_NB = 16
_T = 1024
_NTOK = _NB * _T


def _vq_body(flat_ref, cb_ref, loss_ref, q_ref, perp_ref, enc_ref,
             cnt_ref, acc_ref):
    b = pl.program_id(0)

    @pl.when(b == 0)
    def _init():
        cnt_ref[...] = jnp.zeros_like(cnt_ref)
        acc_ref[0] = 0.0

    fb = flat_ref[...]
    cb = cb_ref[...]

    xsq = jnp.sum(fb * fb, axis=1, keepdims=True)
    cnorm = jnp.sum(cb * cb, axis=1)
    mm = jax.lax.dot_general(fb, cb, (((1,), (1,)), ((), ())),
                             preferred_element_type=jnp.float32)
    d = (xsq + cnorm[None, :]) - 2.0 * mm

    m = jnp.min(d, axis=1, keepdims=True)
    kio = jax.lax.broadcasted_iota(jnp.int32, (_T, _K), 1)
    idx = jnp.min(jnp.where(d == m, kio, _K), axis=1, keepdims=True)

    e = (kio == idx).astype(jnp.float32)
    enc_ref[...] = e
    cnt_ref[...] += jnp.sum(e, axis=0, keepdims=True)

    q = jax.lax.dot_general(e, cb, (((1,), (0,)), ((), ())),
                            preferred_element_type=jnp.float32)
    q_ref[...] = fb + (q - fb)

    acc_ref[0] += jnp.sum(m)

    @pl.when(b == _NB - 1)
    def _fin():
        mse = acc_ref[0] / (_NTOK * _D)
        loss_ref[...] = jnp.reshape((1.0 + _BETA) * mse, (1, 1))
        avg = cnt_ref[...] * (1.0 / _NTOK)
        ent = jnp.sum(avg * jnp.log(avg + 1e-10))
        perp_ref[...] = jnp.reshape(jnp.exp(-ent), (1, 1))


def _vq_call(flat, codebook):
    return pl.pallas_call(
        _vq_body,
        grid=(_NB,),
        in_specs=[
            pl.BlockSpec((_T, _D), lambda b: (b, 0)),
            pl.BlockSpec((_K, _D), lambda b: (0, 0)),
        ],
        out_specs=[
            pl.BlockSpec((1, 1), lambda b: (0, 0)),
            pl.BlockSpec((_T, _D), lambda b: (b, 0)),
            pl.BlockSpec((1, 1), lambda b: (0, 0)),
            pl.BlockSpec((_T, _K), lambda b: (b, 0)),
        ],
        out_shape=[
            jax.ShapeDtypeStruct((1, 1), jnp.float32),
            jax.ShapeDtypeStruct((_NTOK, _D), jnp.float32),
            jax.ShapeDtypeStruct((1, 1), jnp.float32),
            jax.ShapeDtypeStruct((_NTOK, _K), jnp.float32),
        ],
        scratch_shapes=[
            pltpu.VMEM((1, _K), jnp.float32),
            pltpu.SMEM((1,), jnp.float32),
        ],
        compiler_params=pltpu.CompilerParams(
            dimension_semantics=("arbitrary",),
        ),
    )(flat, codebook)


def kernel(encoder_output, codebook):
    b, c, h, w = encoder_output.shape
    flat = jnp.transpose(encoder_output, (0, 2, 3, 1)).reshape(-1, c)
    loss, q_st, perp, enc = _vq_call(flat, codebook)
    q_bchw = jnp.transpose(q_st.reshape(b, h, w, c), (0, 3, 1, 2))
    return (loss[0, 0], q_bchw, perp[0, 0], enc)


# speedup vs baseline: 1.6712x; 1.0786x over previous
import jax
import jax.numpy as jnp
from jax.experimental import pallas as pl
from jax.experimental.pallas import tpu as pltpu

_K = 1024
_D = 256
_BETA = 0.25
_NB = 16
_T = 1024
_NTOK = _NB * _T


def _vq_body(flat_ref, cb_ref, loss_ref, q_ref, perp_ref, enc_ref,
             cnt_ref, acc_ref):
    b = pl.program_id(0)

    @pl.when(b == 0)
    def _init():
        cnt_ref[...] = jnp.zeros_like(cnt_ref)
        acc_ref[0] = 0.0

    fb = flat_ref[...]
    cb = cb_ref[...]

    xsq = jnp.sum(fb * fb, axis=1, keepdims=True)
    cnorm = jnp.sum(cb * cb, axis=1)
    mm = jax.lax.dot_general(fb, cb, (((1,), (1,)), ((), ())),
                             preferred_element_type=jnp.float32)
    d = (xsq + cnorm[None, :]) - 2.0 * mm

    m = jnp.min(d, axis=1, keepdims=True)
    kio = jax.lax.broadcasted_iota(
        jnp.int32, (_T, _K), 1).astype(jnp.float32)
    idx = jnp.min(jnp.where(d == m, kio, float(_K)),
                  axis=1, keepdims=True)

    e = (kio == idx).astype(jnp.float32)
    enc_ref[...] = e
    cnt_ref[...] += jnp.sum(e, axis=0, keepdims=True)

    q = jax.lax.dot_general(e, cb, (((1,), (0,)), ((), ())),
                            preferred_element_type=jnp.float32)
    q_ref[...] = fb + (q - fb)

    acc_ref[0] += jnp.sum(m)

    @pl.when(b == _NB - 1)
    def _fin():
        mse = acc_ref[0] / (_NTOK * _D)
        loss_ref[...] = jnp.reshape((1.0 + _BETA) * mse, (1, 1))
        avg = cnt_ref[...] * (1.0 / _NTOK)
        ent = jnp.sum(avg * jnp.log(avg + 1e-10))
        perp_ref[...] = jnp.reshape(jnp.exp(-ent), (1, 1))


def _vq_call(flat, codebook):
    return pl.pallas_call(
        _vq_body,
        grid=(_NB,),
        in_specs=[
            pl.BlockSpec((_T, _D), lambda b: (b, 0)),
            pl.BlockSpec((_K, _D), lambda b: (0, 0)),
        ],
        out_specs=[
            pl.BlockSpec((1, 1), lambda b: (0, 0)),
            pl.BlockSpec((_T, _D), lambda b: (b, 0)),
            pl.BlockSpec((1, 1), lambda b: (0, 0)),
            pl.BlockSpec((_T, _K), lambda b: (b, 0)),
        ],
        out_shape=[
            jax.ShapeDtypeStruct((1, 1), jnp.float32),
            jax.ShapeDtypeStruct((_NTOK, _D), jnp.float32),
            jax.ShapeDtypeStruct((1, 1), jnp.float32),
            jax.ShapeDtypeStruct((_NTOK, _K), jnp.float32),
        ],
        scratch_shapes=[
            pltpu.VMEM((1, _K), jnp.float32),
            pltpu.SMEM((1,), jnp.float32),
        ],
        compiler_params=pltpu.CompilerParams(
            dimension_semantics=("arbitrary",),
        ),
    )(flat, codebook)


def kernel(encoder_output, codebook):
    b, c, h, w = encoder_output.shape
    flat = jnp.transpose(encoder_output, (0, 2, 3, 1)).reshape(-1, c)
    loss, q_st, perp, enc = _vq_call(flat, codebook)
    q_bchw = jnp.transpose(q_st.reshape(b, h, w, c), (0, 3, 1, 2))
    return (loss[0, 0], q_bchw, perp[0, 0], enc)
